# baseline (speedup 1.0000x reference)
import functools

import jax
import jax.numpy as jnp
from jax.experimental import pallas as pl

N = 8
C_IN = 384
H = 56
W = 56
OUT = 384
N_HASHES = 4
CHUNK = 14
GROUPS = 8
KSIZE = 56
GP = OUT // GROUPS
F_GW = 0.1
F_GV1 = 0.1
F_GV2 = 1.0

B = N * W
COLS = B * H
QKV = OUT * 3 // 2
NB = 4


def _qkv_kernel(w_ref, x_ref, o_ref, sum_ref, sq_ref):
    q = jnp.dot(w_ref[...], x_ref[...], preferred_element_type=jnp.float32)
    o_ref[...] = q
    s = jnp.sum(q, axis=1, keepdims=True)
    s2 = jnp.sum(q * q, axis=1, keepdims=True)

    @pl.when(pl.program_id(0) == 0)
    def _init():
        sum_ref[...] = s
        sq_ref[...] = s2

    @pl.when(pl.program_id(0) != 0)
    def _acc():
        sum_ref[...] += s
        sq_ref[...] += s2


def _qkv_bn(x, conv_w):
    xq = jnp.transpose(x, (1, 0, 3, 2)).reshape(C_IN, COLS)
    bw = 3584
    grid = COLS // bw
    qkv_all, ssum, ssq = pl.pallas_call(
        _qkv_kernel,
        grid=(grid,),
        in_specs=[
            pl.BlockSpec((QKV, C_IN), lambda i: (0, 0)),
            pl.BlockSpec((C_IN, bw), lambda i: (0, i)),
        ],
        out_specs=[
            pl.BlockSpec((QKV, bw), lambda i: (0, i)),
            pl.BlockSpec((QKV, 1), lambda i: (0, 0)),
            pl.BlockSpec((QKV, 1), lambda i: (0, 0)),
        ],
        out_shape=[
            jax.ShapeDtypeStruct((QKV, COLS), jnp.float32),
            jax.ShapeDtypeStruct((QKV, 1), jnp.float32),
            jax.ShapeDtypeStruct((QKV, 1), jnp.float32),
        ],
    )(conv_w, xq)
    mean = ssum[:, 0] / COLS
    var = ssq[:, 0] / COLS - mean * mean
    return qkv_all, mean, var


def kernel(x, conv_w, bn_gamma, bn_beta, relative):
    qkv_all, mean, var = _qkv_bn(x, conv_w)
    scale = bn_gamma / jnp.sqrt(var + 1e-5)
    shift = bn_beta - mean * scale

    qkv = jnp.transpose(qkv_all.reshape(QKV, B, H), (1, 0, 2))
    qkv = qkv * scale[None, :, None] + shift[None, :, None]

    def _normalize(t, axis):
        n = jnp.sqrt(jnp.sum(t * t, axis=axis, keepdims=True))
        return t / jnp.maximum(n, 5e-05)

    def _adj(t):
        return jnp.concatenate(
            [t, jnp.roll(t, 1, axis=3), jnp.roll(t, -1, axis=3)], axis=4)

    def _adj_emb(t):
        return jnp.concatenate(
            [t, jnp.roll(t, 1, axis=0), jnp.roll(t, -1, axis=0)], axis=1)

    qkv2 = qkv.reshape(B, H, QKV)
    w_match = qkv2[:, :, :OUT // 2]
    v_match = qkv2[:, :, OUT // 2:]
    w_embedding = relative[:, :GP // 2]
    v_embedding = relative[:, GP // 2:]
    rot = jax.random.normal(jax.random.key(42), (OUT // 2, N_HASHES, NB // 2),
                            dtype=x.dtype)
    rotated = jnp.einsum('btf,fhi->bhti', w_match, rot)
    rotated = jnp.concatenate([rotated, -rotated], axis=-1)
    hash_codes = jnp.argmax(rotated, axis=-1)
    offsets = (jnp.arange(N_HASHES) * NB).reshape(1, -1, 1)
    hash_codes = (hash_codes + offsets).reshape(B, -1)
    indices = jnp.argsort(hash_codes, axis=-1)
    undo_sort = jnp.argsort(indices, axis=-1)
    mod_indices = indices % H
    w_sorted = jnp.take_along_axis(w_match, mod_indices[:, :, None], axis=1)
    v_sorted = jnp.take_along_axis(v_match, mod_indices[:, :, None], axis=1)
    w_b = w_sorted.reshape(B, GROUPS, N_HASHES, -1, CHUNK, GP // 2)
    v_b = v_sorted.reshape(B, GROUPS, N_HASHES, -1, CHUNK, GP)
    we = w_embedding.reshape(-1, CHUNK, GP // 2)
    ve = v_embedding.reshape(-1, CHUNK, GP)
    w_att_match = _normalize(w_b, -1)
    w_att_match = _adj(w_att_match)
    v_b3 = _adj(v_b)
    we3 = _adj_emb(we)
    ve3 = _adj_emb(ve)
    raw_score = jnp.einsum('bghkie,bghkje->bghkij', w_b, w_att_match)
    w_add = jnp.einsum('bghkie,kje->bghkij', w_b, we3) * F_GW
    ss = jnp.concatenate([raw_score, w_add], axis=1)
    ss = _normalize(ss, 1).reshape(B, 2, GROUPS, N_HASHES, -1, CHUNK,
                                   CHUNK * 3).sum(axis=1)
    bucket_score = jax.scipy.special.logsumexp(ss, axis=-1, keepdims=True)
    score = jnp.exp(ss - bucket_score)
    bucket_score = bucket_score.reshape(B, GROUPS, N_HASHES, -1)
    ret_out = jnp.einsum('bgukij,bgukje->bgukie', score, v_b3) * F_GV2
    ret_emb = jnp.einsum('bgukij,kje->bgukie', score, ve3) * F_GV1
    ret = jnp.concatenate([ret_out, ret_emb], axis=-1).reshape(
        B, N_HASHES, -1, CHUNK, OUT * 2)
    ret = ret.reshape(B, OUT * 2, N_HASHES, -1, CHUNK)
    ret = _normalize(ret, 1).reshape(B, N_HASHES, -1, CHUNK, OUT, 2).sum(-1)
    bucket_score = _normalize(bucket_score, 1).sum(axis=1)
    ret = ret.reshape(B, N_HASHES, -1, OUT)
    ret = ret.reshape(B, -1, OUT)
    bucket_score = bucket_score.reshape(B, -1)
    ret = jnp.take_along_axis(ret, undo_sort[:, :, None], axis=1)
    bucket_score = jnp.take_along_axis(bucket_score, undo_sort, axis=1)
    ret = ret.reshape(B, N_HASHES, H, OUT)
    bucket_score = bucket_score.reshape(B, N_HASHES, H, 1)
    probs = jax.nn.softmax(bucket_score, axis=1)
    ret = jnp.sum(ret * probs, axis=1)
    ret = ret.reshape(N, W, H, OUT)
    return jnp.transpose(ret, (0, 3, 2, 1))


# speedup vs baseline: 3.8234x; 3.8234x over previous
import jax
import jax.numpy as jnp
from jax.experimental import pallas as pl

N = 8
C_IN = 384
H = 56
W = 56
OUT = 384
N_HASHES = 4
CHUNK = 14
GROUPS = 8
KSIZE = 56
GP = OUT // GROUPS
F_GW = 0.1
F_GV1 = 0.1
F_GV2 = 1.0

B = N * W
COLS = B * H
QKV = OUT * 3 // 2
WM = OUT // 2
NB = 4
NEG = -1e30

_INTERPRET = False


def _qkv_kernel(w_ref, x_ref, o_ref, sum_ref, sq_ref):
    q = jnp.dot(w_ref[...], x_ref[...], preferred_element_type=jnp.float32)
    o_ref[...] = q
    s = jnp.sum(q, axis=1, keepdims=True)
    s2 = jnp.sum(q * q, axis=1, keepdims=True)

    @pl.when(pl.program_id(0) == 0)
    def _init():
        sum_ref[...] = s
        sq_ref[...] = s2

    @pl.when(pl.program_id(0) != 0)
    def _acc():
        sum_ref[...] += s
        sq_ref[...] += s2


def _qkv_bn(x, conv_w):
    xq = jnp.transpose(x, (1, 0, 3, 2)).reshape(C_IN, COLS)
    bw = 3584 if COLS % 3584 == 0 else COLS
    grid = COLS // bw
    qkv_all, ssum, ssq = pl.pallas_call(
        _qkv_kernel,
        grid=(grid,),
        in_specs=[
            pl.BlockSpec((QKV, C_IN), lambda i: (0, 0)),
            pl.BlockSpec((C_IN, bw), lambda i: (0, i)),
        ],
        out_specs=[
            pl.BlockSpec((QKV, bw), lambda i: (0, i)),
            pl.BlockSpec((QKV, 1), lambda i: (0, 0)),
            pl.BlockSpec((QKV, 1), lambda i: (0, 0)),
        ],
        out_shape=[
            jax.ShapeDtypeStruct((QKV, COLS), jnp.float32),
            jax.ShapeDtypeStruct((QKV, 1), jnp.float32),
            jax.ShapeDtypeStruct((QKV, 1), jnp.float32),
        ],
        interpret=_INTERPRET,
    )(conv_w, xq)
    mean = ssum[:, 0] / COLS
    var = ssq[:, 0] / COLS - mean * mean
    return qkv_all, mean, var


def _sort_kernel(qkv_ref, sm_ref, sh_ref, rot_ref, sw_ref, sv_ref, pos_ref):
    qn = qkv_ref[0] * sm_ref[...] + sh_ref[...]
    wm = qn[:, :WM]
    rotated = jnp.dot(wm, rot_ref[...],
                      preferred_element_type=jnp.float32)
    rows = jax.lax.broadcasted_iota(jnp.int32, (H, H), 0)
    cols = jax.lax.broadcasted_iota(jnp.int32, (H, H), 1)
    lmat = (rows >= cols).astype(jnp.float32)
    su4 = (jax.lax.broadcasted_iota(jnp.int32, (NB, NB), 0) <
           jax.lax.broadcasted_iota(jnp.int32, (NB, NB), 1)).astype(jnp.float32)
    lane4 = jax.lax.broadcasted_iota(jnp.int32, (H, NB), 1)
    lane56 = jax.lax.broadcasted_iota(jnp.int32, (H, H), 1)
    pos_cols = []
    for h in range(N_HASHES):
        l0 = rotated[:, 2 * h:2 * h + 1]
        l1 = rotated[:, 2 * h + 1:2 * h + 2]
        best = l0
        bi = jnp.zeros((H, 1), jnp.int32)
        for j, v in ((1, l1), (2, -l0), (3, -l1)):
            upd = v > best
            bi = jnp.where(upd, j, bi)
            best = jnp.maximum(best, v)
        onehot = (bi == lane4).astype(jnp.float32)
        csum = jnp.dot(lmat, onehot,
                       preferred_element_type=jnp.float32)
        totals = csum[H - 1:H, :]
        offs = jnp.dot(totals, su4,
                       preferred_element_type=jnp.float32)
        posf = jnp.sum(onehot * (offs + csum), axis=1,
                       keepdims=True) - 1.0
        pos_i = posf.astype(jnp.int32)
        pos_cols.append(pos_i)
        pt = (pos_i == lane56).astype(jnp.float32)
        sorted_h = jax.lax.dot_general(
            pt, qn, (((0,), (0,)), ((), ())),
            preferred_element_type=jnp.float32)
        sw_ref[0, h] = sorted_h[:, :WM]
        sv_ref[0, h] = sorted_h[:, WM:]
    pos_ref[0] = jnp.concatenate(pos_cols, axis=1)


def _sort_gather(qkv2, scale_map, shift_map, rotf):
    return pl.pallas_call(
        _sort_kernel,
        grid=(B,),
        in_specs=[
            pl.BlockSpec((1, H, QKV), lambda i: (i, 0, 0)),
            pl.BlockSpec((H, QKV), lambda i: (0, 0)),
            pl.BlockSpec((H, QKV), lambda i: (0, 0)),
            pl.BlockSpec((WM, 2 * N_HASHES), lambda i: (0, 0)),
        ],
        out_specs=[
            pl.BlockSpec((1, N_HASHES, H, WM), lambda i: (i, 0, 0, 0)),
            pl.BlockSpec((1, N_HASHES, H, OUT), lambda i: (i, 0, 0, 0)),
            pl.BlockSpec((1, H, N_HASHES), lambda i: (i, 0, 0)),
        ],
        out_shape=[
            jax.ShapeDtypeStruct((B, N_HASHES, H, WM), jnp.float32),
            jax.ShapeDtypeStruct((B, N_HASHES, H, OUT), jnp.float32),
            jax.ShapeDtypeStruct((B, H, N_HASHES), jnp.int32),
        ],
        interpret=_INTERPRET,
    )(qkv2, scale_map, shift_map, rotf)


def _attn_kernel(ws_ref, vs_ref, we_ref, ve_ref, out_ref, bsn_ref):
    we = we_ref[...]
    ve = ve_ref[...]
    ci = jax.lax.broadcasted_iota(jnp.int32, (H, H), 0) // CHUNK
    cj = jax.lax.broadcasted_iota(jnp.int32, (H, H), 1) // CHUNK
    allowed = cj != ((ci + 2) & 3)

    s_mat = {}
    a_mat = {}
    for h in range(N_HASHES):
        wh = ws_ref[0, h]
        for au in range(8):
            a, u = au // 4, au % 4
            g = 2 * h + a
            q = wh[au * H:(au + 1) * H, :]
            nrm = jnp.sqrt(jnp.sum(q * q, axis=1, keepdims=True))
            qn = q / jnp.maximum(nrm, 5e-5)
            s_mat[(g, u)] = jax.lax.dot_general(
                q, qn, (((1,), (1,)), ((), ())),
                preferred_element_type=jnp.float32)
            a_mat[(g, u)] = jax.lax.dot_general(
                q, we, (((1,), (1,)), ((), ())),
                preferred_element_type=jnp.float32) * F_GW

    r96 = [None] * (GROUPS * N_HASHES)
    bs = {}
    for u in range(N_HASHES):
        sumsq = jnp.zeros((H, H), jnp.float32)
        for g in range(GROUPS):
            sumsq += s_mat[(g, u)] * s_mat[(g, u)]
            sumsq += a_mat[(g, u)] * a_mat[(g, u)]
        den = jnp.maximum(jnp.sqrt(sumsq), 5e-5)
        for g in range(GROUPS):
            ssm = jnp.where(allowed,
                            (s_mat[(g, u)] + a_mat[(g, u)]) / den, NEG)
            m = jnp.max(ssm, axis=1, keepdims=True)
            e = jnp.exp(ssm - m)
            se = jnp.sum(e, axis=1, keepdims=True)
            bs[(g, u)] = m + jnp.log(se)
            p = e / se
            h2, a2 = g // 2, g % 2
            v = vs_ref[0, h2][(a2 * 4 + u) * H:(a2 * 4 + u + 1) * H, :]
            rout = jnp.dot(p, v, preferred_element_type=jnp.float32) * F_GV2
            remb = jnp.dot(p, ve, preferred_element_type=jnp.float32) * F_GV1
            r96[g * N_HASHES + u] = jnp.concatenate([rout, remb], axis=1)

    out_ref[0] = jnp.concatenate(r96, axis=0)

    bsn = []
    for u in range(N_HASHES):
        sq = jnp.zeros((H, 1), jnp.float32)
        for g in range(GROUPS):
            sq += bs[(g, u)] * bs[(g, u)]
        nb = jnp.maximum(jnp.sqrt(sq), 5e-5)
        acc = jnp.zeros((H, 1), jnp.float32)
        for g in range(GROUPS):
            acc += bs[(g, u)] / nb
        bsn.append(acc)
    bsn_ref[0] = jnp.concatenate(bsn, axis=1)


def _attention(ws_r, vs_r, we_full, ve_full):
    return pl.pallas_call(
        _attn_kernel,
        grid=(B,),
        in_specs=[
            pl.BlockSpec((1, N_HASHES, 8 * H, GP // 2), lambda i: (i, 0, 0, 0)),
            pl.BlockSpec((1, N_HASHES, 8 * H, GP), lambda i: (i, 0, 0, 0)),
            pl.BlockSpec((KSIZE, GP // 2), lambda i: (0, 0)),
            pl.BlockSpec((KSIZE, GP), lambda i: (0, 0)),
        ],
        out_specs=[
            pl.BlockSpec((1, 32 * H, 2 * GP), lambda i: (i, 0, 0)),
            pl.BlockSpec((1, H, N_HASHES), lambda i: (i, 0, 0)),
        ],
        out_shape=[
            jax.ShapeDtypeStruct((B, 32 * H, 2 * GP), jnp.float32),
            jax.ShapeDtypeStruct((B, H, N_HASHES), jnp.float32),
        ],
        interpret=_INTERPRET,
    )(ws_r, vs_r, we_full, ve_full)


def _cnorm_kernel(x_ref, o_ref):
    x = x_ref[...]
    nrm = jnp.sqrt(jnp.sum(x * x, axis=1, keepdims=True))
    xn = x / jnp.maximum(nrm, 5e-5)
    o_ref[...] = xn + jnp.roll(xn, -1, axis=2)


def _cnorm(xmat):
    bb = 8
    return pl.pallas_call(
        _cnorm_kernel,
        grid=(B // bb,),
        in_specs=[pl.BlockSpec((bb, 2 * OUT, N_HASHES * H),
                               lambda i: (i, 0, 0))],
        out_specs=pl.BlockSpec((bb, 2 * OUT, N_HASHES * H),
                               lambda i: (i, 0, 0)),
        out_shape=jax.ShapeDtypeStruct((B, 2 * OUT, N_HASHES * H),
                                       jnp.float32),
        interpret=_INTERPRET,
    )(xmat)


def _combine_kernel(y_ref, bsn_ref, pos_ref, out_ref):
    lane56 = jax.lax.broadcasted_iota(jnp.int32, (H, H), 1)
    ret_u = []
    bs_u = []
    for h in range(N_HASHES):
        pos_i = pos_ref[0][:, h:h + 1]
        pt = (pos_i == lane56).astype(jnp.float32)
        ret_u.append(jnp.dot(pt, y_ref[0][h * H:(h + 1) * H, :],
                             preferred_element_type=jnp.float32))
        bs_u.append(jnp.dot(pt, bsn_ref[0][:, h:h + 1],
                            preferred_element_type=jnp.float32))
    bsc = jnp.concatenate(bs_u, axis=1)
    m = jnp.max(bsc, axis=1, keepdims=True)
    e = jnp.exp(bsc - m)
    probs = e / jnp.sum(e, axis=1, keepdims=True)
    out = jnp.zeros((H, OUT), jnp.float32)
    for h in range(N_HASHES):
        out += probs[:, h:h + 1] * ret_u[h]
    out_ref[0] = out


def _combine(o_all, bsn, pos):
    return pl.pallas_call(
        _combine_kernel,
        grid=(B,),
        in_specs=[
            pl.BlockSpec((1, N_HASHES * H, OUT), lambda i: (i, 0, 0)),
            pl.BlockSpec((1, H, N_HASHES), lambda i: (i, 0, 0)),
            pl.BlockSpec((1, H, N_HASHES), lambda i: (i, 0, 0)),
        ],
        out_specs=pl.BlockSpec((1, H, OUT), lambda i: (i, 0, 0)),
        out_shape=jax.ShapeDtypeStruct((B, H, OUT), jnp.float32),
        interpret=_INTERPRET,
    )(o_all, bsn, pos)


def kernel(x, conv_w, bn_gamma, bn_beta, relative):
    qkv_all, mean, var = _qkv_bn(x, conv_w)
    scale = bn_gamma / jnp.sqrt(var + 1e-5)
    shift = bn_beta - mean * scale

    qkv2 = jnp.transpose(qkv_all.reshape(QKV, B, H), (1, 0, 2)).reshape(
        B, H, QKV)
    scale_map = jnp.broadcast_to(scale[:, None], (QKV, H)).reshape(H, QKV)
    shift_map = jnp.broadcast_to(shift[:, None], (QKV, H)).reshape(H, QKV)
    rotf = jax.random.normal(jax.random.key(42),
                             (WM, N_HASHES, NB // 2),
                             dtype=x.dtype).reshape(WM, 2 * N_HASHES)

    sorted_w, sorted_v, pos = _sort_gather(qkv2, scale_map, shift_map, rotf)
    ws_r = sorted_w.reshape(B, N_HASHES, 8 * H, GP // 2)
    vs_r = sorted_v.reshape(B, N_HASHES, 8 * H, GP)
    we_full = relative[:, :GP // 2]
    ve_full = relative[:, GP // 2:]

    rbig, bsn = _attention(ws_r, vs_r, we_full, ve_full)
    xmat = rbig.reshape(B, 2 * OUT, N_HASHES * H)
    spair = _cnorm(xmat)
    o_all = spair.reshape(B, N_HASHES * H, OUT, 2)[..., 0]
    ret = _combine(o_all, bsn, pos)
    ret = ret.reshape(N, W, H, OUT)
    return jnp.transpose(ret, (0, 3, 2, 1))


# speedup vs baseline: 3.8866x; 1.0165x over previous
import jax
import jax.numpy as jnp
from jax.experimental import pallas as pl

N = 8
C_IN = 384
H = 56
W = 56
OUT = 384
N_HASHES = 4
CHUNK = 14
GROUPS = 8
KSIZE = 56
GP = OUT // GROUPS
F_GW = 0.1
F_GV1 = 0.1
F_GV2 = 1.0

B = N * W
COLS = B * H
QKV = OUT * 3 // 2
WM = OUT // 2
NB = 4
NEG = -1e30

_INTERPRET = False


def _qkv_kernel(w_ref, x_ref, o_ref, sum_ref, sq_ref):
    q = jnp.dot(w_ref[...], x_ref[...], preferred_element_type=jnp.float32)
    o_ref[...] = q
    s = jnp.sum(q, axis=1, keepdims=True)
    s2 = jnp.sum(q * q, axis=1, keepdims=True)

    @pl.when(pl.program_id(0) == 0)
    def _init():
        sum_ref[...] = s
        sq_ref[...] = s2

    @pl.when(pl.program_id(0) != 0)
    def _acc():
        sum_ref[...] += s
        sq_ref[...] += s2


def _qkv_bn(x, conv_w):
    xq = jnp.transpose(x, (1, 0, 3, 2)).reshape(C_IN, COLS)
    bw = 3584 if COLS % 3584 == 0 else COLS
    grid = COLS // bw
    qkv_all, ssum, ssq = pl.pallas_call(
        _qkv_kernel,
        grid=(grid,),
        in_specs=[
            pl.BlockSpec((QKV, C_IN), lambda i: (0, 0)),
            pl.BlockSpec((C_IN, bw), lambda i: (0, i)),
        ],
        out_specs=[
            pl.BlockSpec((QKV, bw), lambda i: (0, i)),
            pl.BlockSpec((QKV, 1), lambda i: (0, 0)),
            pl.BlockSpec((QKV, 1), lambda i: (0, 0)),
        ],
        out_shape=[
            jax.ShapeDtypeStruct((QKV, COLS), jnp.float32),
            jax.ShapeDtypeStruct((QKV, 1), jnp.float32),
            jax.ShapeDtypeStruct((QKV, 1), jnp.float32),
        ],
        interpret=_INTERPRET,
    )(conv_w, xq)
    mean = ssum[:, 0] / COLS
    var = ssq[:, 0] / COLS - mean * mean
    return qkv_all, mean, var


def _sort_kernel(qkv_ref, sm_ref, sh_ref, rot_ref, sw_ref, sv_ref, pos_ref):
    qn = qkv_ref[0] * sm_ref[...] + sh_ref[...]
    wm = qn[:, :WM]
    rotated = jnp.dot(wm, rot_ref[...],
                      preferred_element_type=jnp.float32)
    rows = jax.lax.broadcasted_iota(jnp.int32, (H, H), 0)
    cols = jax.lax.broadcasted_iota(jnp.int32, (H, H), 1)
    lmat = (rows >= cols).astype(jnp.float32)
    su4 = (jax.lax.broadcasted_iota(jnp.int32, (NB, NB), 0) <
           jax.lax.broadcasted_iota(jnp.int32, (NB, NB), 1)).astype(jnp.float32)
    lane4 = jax.lax.broadcasted_iota(jnp.int32, (H, NB), 1)
    lane56 = jax.lax.broadcasted_iota(jnp.int32, (H, H), 1)
    pos_cols = []
    for h in range(N_HASHES):
        l0 = rotated[:, 2 * h:2 * h + 1]
        l1 = rotated[:, 2 * h + 1:2 * h + 2]
        best = l0
        bi = jnp.zeros((H, 1), jnp.int32)
        for j, v in ((1, l1), (2, -l0), (3, -l1)):
            upd = v > best
            bi = jnp.where(upd, j, bi)
            best = jnp.maximum(best, v)
        onehot = (bi == lane4).astype(jnp.float32)
        csum = jnp.dot(lmat, onehot,
                       preferred_element_type=jnp.float32)
        totals = csum[H - 1:H, :]
        offs = jnp.dot(totals, su4,
                       preferred_element_type=jnp.float32)
        posf = jnp.sum(onehot * (offs + csum), axis=1,
                       keepdims=True) - 1.0
        pos_i = posf.astype(jnp.int32)
        pos_cols.append(pos_i)
        pt = (pos_i == lane56).astype(jnp.float32)
        sorted_h = jax.lax.dot_general(
            pt, qn, (((0,), (0,)), ((), ())),
            preferred_element_type=jnp.float32)
        sw_ref[0, h] = sorted_h[:, :WM]
        sv_ref[0, h] = sorted_h[:, WM:]
    pos_ref[0] = jnp.concatenate(pos_cols, axis=1)


def _sort_gather(qkv2, scale_map, shift_map, rotf):
    return pl.pallas_call(
        _sort_kernel,
        grid=(B,),
        in_specs=[
            pl.BlockSpec((1, H, QKV), lambda i: (i, 0, 0)),
            pl.BlockSpec((H, QKV), lambda i: (0, 0)),
            pl.BlockSpec((H, QKV), lambda i: (0, 0)),
            pl.BlockSpec((WM, 2 * N_HASHES), lambda i: (0, 0)),
        ],
        out_specs=[
            pl.BlockSpec((1, N_HASHES, H, WM), lambda i: (i, 0, 0, 0)),
            pl.BlockSpec((1, N_HASHES, H, OUT), lambda i: (i, 0, 0, 0)),
            pl.BlockSpec((1, H, N_HASHES), lambda i: (i, 0, 0)),
        ],
        out_shape=[
            jax.ShapeDtypeStruct((B, N_HASHES, H, WM), jnp.float32),
            jax.ShapeDtypeStruct((B, N_HASHES, H, OUT), jnp.float32),
            jax.ShapeDtypeStruct((B, H, N_HASHES), jnp.int32),
        ],
        interpret=_INTERPRET,
    )(qkv2, scale_map, shift_map, rotf)


def _attn_kernel(ws_ref, vs_ref, we_ref, ve_ref, out_ref, bsn_ref):
    wes = we_ref[...] * F_GW
    ves = ve_ref[...] * F_GV1
    ci = jax.lax.broadcasted_iota(jnp.int32, (H, H), 0) // CHUNK
    cj = jax.lax.broadcasted_iota(jnp.int32, (H, H), 1) // CHUNK
    allowed = cj != ((ci + 2) & 3)

    qstack = ws_ref[0].reshape(8 * N_HASHES * H, GP // 2)
    nrm = jnp.sqrt(jnp.sum(qstack * qstack, axis=1, keepdims=True))
    qn_stack = qstack / jnp.maximum(nrm, 5e-5)

    def _blk(arr, h, au):
        base = h * 8 * H + au * H
        return arr[base:base + H, :]

    sa_list = []
    for g in range(GROUPS):
        h, a = g // 2, g % 2
        for u in range(N_HASHES):
            au = a * 4 + u
            q = _blk(qstack, h, au)
            rhs = jnp.concatenate([_blk(qn_stack, h, au), wes], axis=0)
            sa_list.append(jax.lax.dot_general(
                q, rhs, (((1,), (1,)), ((), ())),
                preferred_element_type=jnp.float32))
    sa = jnp.concatenate(sa_list, axis=0).reshape(GROUPS, N_HASHES, H, 2 * H)
    ssq = jnp.sum(sa * sa, axis=0)
    den = jnp.maximum(jnp.sqrt(ssq[:, :, :H] + ssq[:, :, H:]), 5e-5)
    ss = (sa[..., :H] + sa[..., H:]) / den[None]
    ssm = jnp.where(allowed[None, None], ss, NEG)
    m = jnp.max(ssm, axis=-1, keepdims=True)
    e = jnp.exp(ssm - m)
    se = jnp.sum(e, axis=-1, keepdims=True)
    bs_all = m + jnp.log(se)
    p_all = e / se

    r96 = []
    for g in range(GROUPS):
        h, a = g // 2, g % 2
        for u in range(N_HASHES):
            au = a * 4 + u
            vcat = jnp.concatenate(
                [vs_ref[0, h][au * H:(au + 1) * H, :], ves], axis=1)
            r96.append(jnp.dot(p_all[g, u], vcat,
                               preferred_element_type=jnp.float32))
    out_ref[0] = jnp.concatenate(r96, axis=0)

    sq = jnp.sum(bs_all * bs_all, axis=0)
    nb = jnp.maximum(jnp.sqrt(sq), 5e-5)
    acc = jnp.sum(bs_all, axis=0) / nb
    bsn_ref[0] = jnp.concatenate([acc[u] for u in range(N_HASHES)], axis=1)


def _attention(ws_r, vs_r, we_full, ve_full):
    return pl.pallas_call(
        _attn_kernel,
        grid=(B,),
        in_specs=[
            pl.BlockSpec((1, N_HASHES, 8 * H, GP // 2), lambda i: (i, 0, 0, 0)),
            pl.BlockSpec((1, N_HASHES, 8 * H, GP), lambda i: (i, 0, 0, 0)),
            pl.BlockSpec((KSIZE, GP // 2), lambda i: (0, 0)),
            pl.BlockSpec((KSIZE, GP), lambda i: (0, 0)),
        ],
        out_specs=[
            pl.BlockSpec((1, 32 * H, 2 * GP), lambda i: (i, 0, 0)),
            pl.BlockSpec((1, H, N_HASHES), lambda i: (i, 0, 0)),
        ],
        out_shape=[
            jax.ShapeDtypeStruct((B, 32 * H, 2 * GP), jnp.float32),
            jax.ShapeDtypeStruct((B, H, N_HASHES), jnp.float32),
        ],
        interpret=_INTERPRET,
    )(ws_r, vs_r, we_full, ve_full)


def _cnorm_kernel(x_ref, o_ref):
    x = x_ref[...]
    nrm = jnp.sqrt(jnp.sum(x * x, axis=1, keepdims=True))
    xn = x / jnp.maximum(nrm, 5e-5)
    o_ref[...] = xn + jnp.roll(xn, -1, axis=2)


def _cnorm(xmat):
    bb = 8
    return pl.pallas_call(
        _cnorm_kernel,
        grid=(B // bb,),
        in_specs=[pl.BlockSpec((bb, 2 * OUT, N_HASHES * H),
                               lambda i: (i, 0, 0))],
        out_specs=pl.BlockSpec((bb, 2 * OUT, N_HASHES * H),
                               lambda i: (i, 0, 0)),
        out_shape=jax.ShapeDtypeStruct((B, 2 * OUT, N_HASHES * H),
                                       jnp.float32),
        interpret=_INTERPRET,
    )(xmat)


def _combine_kernel(y_ref, bsn_ref, pos_ref, out_ref):
    lane56 = jax.lax.broadcasted_iota(jnp.int32, (H, H), 1)
    ret_u = []
    bs_u = []
    for h in range(N_HASHES):
        pos_i = pos_ref[0][:, h:h + 1]
        pt = (pos_i == lane56).astype(jnp.float32)
        ret_u.append(jnp.dot(pt, y_ref[0][h * H:(h + 1) * H, :],
                             preferred_element_type=jnp.float32))
        bs_u.append(jnp.dot(pt, bsn_ref[0][:, h:h + 1],
                            preferred_element_type=jnp.float32))
    bsc = jnp.concatenate(bs_u, axis=1)
    m = jnp.max(bsc, axis=1, keepdims=True)
    e = jnp.exp(bsc - m)
    probs = e / jnp.sum(e, axis=1, keepdims=True)
    out = jnp.zeros((H, OUT), jnp.float32)
    for h in range(N_HASHES):
        out += probs[:, h:h + 1] * ret_u[h]
    out_ref[0] = out


def _combine(o_all, bsn, pos):
    return pl.pallas_call(
        _combine_kernel,
        grid=(B,),
        in_specs=[
            pl.BlockSpec((1, N_HASHES * H, OUT), lambda i: (i, 0, 0)),
            pl.BlockSpec((1, H, N_HASHES), lambda i: (i, 0, 0)),
            pl.BlockSpec((1, H, N_HASHES), lambda i: (i, 0, 0)),
        ],
        out_specs=pl.BlockSpec((1, H, OUT), lambda i: (i, 0, 0)),
        out_shape=jax.ShapeDtypeStruct((B, H, OUT), jnp.float32),
        interpret=_INTERPRET,
    )(o_all, bsn, pos)


def kernel(x, conv_w, bn_gamma, bn_beta, relative):
    qkv_all, mean, var = _qkv_bn(x, conv_w)
    scale = bn_gamma / jnp.sqrt(var + 1e-5)
    shift = bn_beta - mean * scale

    qkv2 = jnp.transpose(qkv_all.reshape(QKV, B, H), (1, 0, 2)).reshape(
        B, H, QKV)
    scale_map = jnp.broadcast_to(scale[:, None], (QKV, H)).reshape(H, QKV)
    shift_map = jnp.broadcast_to(shift[:, None], (QKV, H)).reshape(H, QKV)
    rotf = jax.random.normal(jax.random.key(42),
                             (WM, N_HASHES, NB // 2),
                             dtype=x.dtype).reshape(WM, 2 * N_HASHES)

    sorted_w, sorted_v, pos = _sort_gather(qkv2, scale_map, shift_map, rotf)
    ws_r = sorted_w.reshape(B, N_HASHES, 8 * H, GP // 2)
    vs_r = sorted_v.reshape(B, N_HASHES, 8 * H, GP)
    we_full = relative[:, :GP // 2]
    ve_full = relative[:, GP // 2:]

    rbig, bsn = _attention(ws_r, vs_r, we_full, ve_full)
    xmat = rbig.reshape(B, 2 * OUT, N_HASHES * H)
    spair = _cnorm(xmat)
    o_all = spair.reshape(B, N_HASHES * H, OUT, 2)[..., 0]
    ret = _combine(o_all, bsn, pos)
    ret = ret.reshape(N, W, H, OUT)
    return jnp.transpose(ret, (0, 3, 2, 1))


# speedup vs baseline: 4.0967x; 1.0541x over previous
import jax
import jax.numpy as jnp
from jax.experimental import pallas as pl

N = 8
C_IN = 384
H = 56
W = 56
OUT = 384
N_HASHES = 4
CHUNK = 14
GROUPS = 8
KSIZE = 56
GP = OUT // GROUPS
F_GW = 0.1
F_GV1 = 0.1
F_GV2 = 1.0

B = N * W
COLS = B * H
QKV = OUT * 3 // 2
WM = OUT // 2
NB = 4
NEG = -1e30

_INTERPRET = False


def _qkv_kernel(w_ref, x_ref, o_ref, sum_ref, sq_ref):
    q = jnp.dot(w_ref[...], x_ref[...], preferred_element_type=jnp.float32)
    o_ref[...] = q
    s = jnp.sum(q, axis=1, keepdims=True)
    s2 = jnp.sum(q * q, axis=1, keepdims=True)

    @pl.when(pl.program_id(0) == 0)
    def _init():
        sum_ref[...] = s
        sq_ref[...] = s2

    @pl.when(pl.program_id(0) != 0)
    def _acc():
        sum_ref[...] += s
        sq_ref[...] += s2


def _qkv_bn(x, conv_w):
    xq = jnp.transpose(x, (1, 0, 3, 2)).reshape(C_IN, COLS)
    bw = 3584 if COLS % 3584 == 0 else COLS
    grid = COLS // bw
    qkv_all, ssum, ssq = pl.pallas_call(
        _qkv_kernel,
        grid=(grid,),
        in_specs=[
            pl.BlockSpec((QKV, C_IN), lambda i: (0, 0)),
            pl.BlockSpec((C_IN, bw), lambda i: (0, i)),
        ],
        out_specs=[
            pl.BlockSpec((QKV, bw), lambda i: (0, i)),
            pl.BlockSpec((QKV, 1), lambda i: (0, 0)),
            pl.BlockSpec((QKV, 1), lambda i: (0, 0)),
        ],
        out_shape=[
            jax.ShapeDtypeStruct((QKV, COLS), jnp.float32),
            jax.ShapeDtypeStruct((QKV, 1), jnp.float32),
            jax.ShapeDtypeStruct((QKV, 1), jnp.float32),
        ],
        interpret=_INTERPRET,
    )(conv_w, xq)
    mean = ssum[:, 0] / COLS
    var = ssq[:, 0] / COLS - mean * mean
    return qkv_all, mean, var


BBS = 4


def _sort_kernel(qkv_ref, sm_ref, sh_ref, rot_ref, sw_ref, sv_ref, pos_ref):
    rows = jax.lax.broadcasted_iota(jnp.int32, (H, H), 0)
    cols = jax.lax.broadcasted_iota(jnp.int32, (H, H), 1)
    lmat = (rows >= cols).astype(jnp.float32)
    su4 = (jax.lax.broadcasted_iota(jnp.int32, (NB, NB), 0) <
           jax.lax.broadcasted_iota(jnp.int32, (NB, NB), 1)).astype(jnp.float32)
    lane4 = jax.lax.broadcasted_iota(jnp.int32, (H, NB), 1)
    lane56 = jax.lax.broadcasted_iota(jnp.int32, (H, H), 1)
    sm = sm_ref[...]
    sh = sh_ref[...]
    rot = rot_ref[...]
    for lb in range(BBS):
        qn = qkv_ref[lb] * sm + sh
        wm = qn[:, :WM]
        rotated = jnp.dot(wm, rot,
                          preferred_element_type=jnp.float32)
        pos_cols = []
        for h in range(N_HASHES):
            l0 = rotated[:, 2 * h:2 * h + 1]
            l1 = rotated[:, 2 * h + 1:2 * h + 2]
            best = l0
            bi = jnp.zeros((H, 1), jnp.int32)
            for j, v in ((1, l1), (2, -l0), (3, -l1)):
                upd = v > best
                bi = jnp.where(upd, j, bi)
                best = jnp.maximum(best, v)
            onehot = (bi == lane4).astype(jnp.float32)
            csum = jnp.dot(lmat, onehot,
                           preferred_element_type=jnp.float32)
            totals = csum[H - 1:H, :]
            offs = jnp.dot(totals, su4,
                           preferred_element_type=jnp.float32)
            posf = jnp.sum(onehot * (offs + csum), axis=1,
                           keepdims=True) - 1.0
            pos_i = posf.astype(jnp.int32)
            pos_cols.append(pos_i)
            pt = (pos_i == lane56).astype(jnp.float32)
            sorted_h = jax.lax.dot_general(
                pt, qn, (((0,), (0,)), ((), ())),
                preferred_element_type=jnp.float32)
            sw_ref[lb, h] = sorted_h[:, :WM]
            sv_ref[lb, h] = sorted_h[:, WM:]
        pos_ref[lb] = jnp.concatenate(pos_cols, axis=1)


def _sort_gather(qkv2, scale_map, shift_map, rotf):
    return pl.pallas_call(
        _sort_kernel,
        grid=(B // BBS,),
        in_specs=[
            pl.BlockSpec((BBS, H, QKV), lambda i: (i, 0, 0)),
            pl.BlockSpec((H, QKV), lambda i: (0, 0)),
            pl.BlockSpec((H, QKV), lambda i: (0, 0)),
            pl.BlockSpec((WM, 2 * N_HASHES), lambda i: (0, 0)),
        ],
        out_specs=[
            pl.BlockSpec((BBS, N_HASHES, H, WM), lambda i: (i, 0, 0, 0)),
            pl.BlockSpec((BBS, N_HASHES, H, OUT), lambda i: (i, 0, 0, 0)),
            pl.BlockSpec((BBS, H, N_HASHES), lambda i: (i, 0, 0)),
        ],
        out_shape=[
            jax.ShapeDtypeStruct((B, N_HASHES, H, WM), jnp.float32),
            jax.ShapeDtypeStruct((B, N_HASHES, H, OUT), jnp.float32),
            jax.ShapeDtypeStruct((B, H, N_HASHES), jnp.int32),
        ],
        interpret=_INTERPRET,
    )(qkv2, scale_map, shift_map, rotf)


def _attn_kernel(ws_ref, vs_ref, we_ref, ve_ref, out_ref, bsn_ref):
    wes = we_ref[...] * F_GW
    ves = ve_ref[...] * F_GV1
    ci = jax.lax.broadcasted_iota(jnp.int32, (H, H), 0) // CHUNK
    cj = jax.lax.broadcasted_iota(jnp.int32, (H, H), 1) // CHUNK
    allowed = cj != ((ci + 2) & 3)

    qstack = ws_ref[0].reshape(8 * N_HASHES * H, GP // 2)
    nrm = jnp.sqrt(jnp.sum(qstack * qstack, axis=1, keepdims=True))
    qn_stack = qstack / jnp.maximum(nrm, 5e-5)

    def _blk(arr, h, au):
        base = h * 8 * H + au * H
        return arr[base:base + H, :]

    sa_list = []
    for g in range(GROUPS):
        h, a = g // 2, g % 2
        for u in range(N_HASHES):
            au = a * 4 + u
            q = _blk(qstack, h, au)
            rhs = jnp.concatenate([_blk(qn_stack, h, au), wes], axis=0)
            sa_list.append(jax.lax.dot_general(
                q, rhs, (((1,), (1,)), ((), ())),
                preferred_element_type=jnp.float32))
    sa = jnp.concatenate(sa_list, axis=0).reshape(GROUPS, N_HASHES, H, 2 * H)
    ssq = jnp.sum(sa * sa, axis=0)
    den = jnp.maximum(jnp.sqrt(ssq[:, :, :H] + ssq[:, :, H:]), 5e-5)
    ss = (sa[..., :H] + sa[..., H:]) / den[None]
    ssm = jnp.where(allowed[None, None], ss, NEG)
    m = jnp.max(ssm, axis=-1, keepdims=True)
    e = jnp.exp(ssm - m)
    se = jnp.sum(e, axis=-1, keepdims=True)
    bs_all = m + jnp.log(se)
    p_all = e / se

    r96 = []
    for g in range(GROUPS):
        h, a = g // 2, g % 2
        for u in range(N_HASHES):
            au = a * 4 + u
            vcat = jnp.concatenate(
                [vs_ref[0, h][au * H:(au + 1) * H, :], ves], axis=1)
            r96.append(jnp.dot(p_all[g, u], vcat,
                               preferred_element_type=jnp.float32))
    out_ref[0] = jnp.concatenate(r96, axis=0)

    sq = jnp.sum(bs_all * bs_all, axis=0)
    nb = jnp.maximum(jnp.sqrt(sq), 5e-5)
    acc = jnp.sum(bs_all, axis=0) / nb
    bsn_ref[0] = jnp.concatenate([acc[u] for u in range(N_HASHES)], axis=1)


def _attention(ws_r, vs_r, we_full, ve_full):
    return pl.pallas_call(
        _attn_kernel,
        grid=(B,),
        in_specs=[
            pl.BlockSpec((1, N_HASHES, 8 * H, GP // 2), lambda i: (i, 0, 0, 0)),
            pl.BlockSpec((1, N_HASHES, 8 * H, GP), lambda i: (i, 0, 0, 0)),
            pl.BlockSpec((KSIZE, GP // 2), lambda i: (0, 0)),
            pl.BlockSpec((KSIZE, GP), lambda i: (0, 0)),
        ],
        out_specs=[
            pl.BlockSpec((1, 32 * H, 2 * GP), lambda i: (i, 0, 0)),
            pl.BlockSpec((1, H, N_HASHES), lambda i: (i, 0, 0)),
        ],
        out_shape=[
            jax.ShapeDtypeStruct((B, 32 * H, 2 * GP), jnp.float32),
            jax.ShapeDtypeStruct((B, H, N_HASHES), jnp.float32),
        ],
        interpret=_INTERPRET,
    )(ws_r, vs_r, we_full, ve_full)


def _cnorm_kernel(x_ref, o_ref):
    x = x_ref[...]
    nrm = jnp.sqrt(jnp.sum(x * x, axis=1, keepdims=True))
    xn = x / jnp.maximum(nrm, 5e-5)
    o_ref[...] = xn + jnp.roll(xn, -1, axis=2)


def _cnorm(xmat):
    bb = 8
    return pl.pallas_call(
        _cnorm_kernel,
        grid=(B // bb,),
        in_specs=[pl.BlockSpec((bb, 2 * OUT, N_HASHES * H),
                               lambda i: (i, 0, 0))],
        out_specs=pl.BlockSpec((bb, 2 * OUT, N_HASHES * H),
                               lambda i: (i, 0, 0)),
        out_shape=jax.ShapeDtypeStruct((B, 2 * OUT, N_HASHES * H),
                                       jnp.float32),
        interpret=_INTERPRET,
    )(xmat)


def _combine_kernel(y_ref, bsn_ref, pos_ref, out_ref):
    lane56 = jax.lax.broadcasted_iota(jnp.int32, (H, H), 1)
    for lb in range(BBS):
        ret_u = []
        bs_u = []
        for h in range(N_HASHES):
            pos_i = pos_ref[lb][:, h:h + 1]
            pt = (pos_i == lane56).astype(jnp.float32)
            ret_u.append(jnp.dot(pt, y_ref[lb][h * H:(h + 1) * H, :],
                                 preferred_element_type=jnp.float32))
            bs_u.append(jnp.dot(pt, bsn_ref[lb][:, h:h + 1],
                                preferred_element_type=jnp.float32))
        bsc = jnp.concatenate(bs_u, axis=1)
        m = jnp.max(bsc, axis=1, keepdims=True)
        e = jnp.exp(bsc - m)
        probs = e / jnp.sum(e, axis=1, keepdims=True)
        out = jnp.zeros((H, OUT), jnp.float32)
        for h in range(N_HASHES):
            out += probs[:, h:h + 1] * ret_u[h]
        out_ref[lb] = out


def _combine(o_all, bsn, pos):
    return pl.pallas_call(
        _combine_kernel,
        grid=(B // BBS,),
        in_specs=[
            pl.BlockSpec((BBS, N_HASHES * H, OUT), lambda i: (i, 0, 0)),
            pl.BlockSpec((BBS, H, N_HASHES), lambda i: (i, 0, 0)),
            pl.BlockSpec((BBS, H, N_HASHES), lambda i: (i, 0, 0)),
        ],
        out_specs=pl.BlockSpec((BBS, H, OUT), lambda i: (i, 0, 0)),
        out_shape=jax.ShapeDtypeStruct((B, H, OUT), jnp.float32),
        interpret=_INTERPRET,
    )(o_all, bsn, pos)


def kernel(x, conv_w, bn_gamma, bn_beta, relative):
    qkv_all, mean, var = _qkv_bn(x, conv_w)
    scale = bn_gamma / jnp.sqrt(var + 1e-5)
    shift = bn_beta - mean * scale

    qkv2 = jnp.transpose(qkv_all.reshape(QKV, B, H), (1, 0, 2)).reshape(
        B, H, QKV)
    scale_map = jnp.broadcast_to(scale[:, None], (QKV, H)).reshape(H, QKV)
    shift_map = jnp.broadcast_to(shift[:, None], (QKV, H)).reshape(H, QKV)
    rotf = jax.random.normal(jax.random.key(42),
                             (WM, N_HASHES, NB // 2),
                             dtype=x.dtype).reshape(WM, 2 * N_HASHES)

    sorted_w, sorted_v, pos = _sort_gather(qkv2, scale_map, shift_map, rotf)
    ws_r = sorted_w.reshape(B, N_HASHES, 8 * H, GP // 2)
    vs_r = sorted_v.reshape(B, N_HASHES, 8 * H, GP)
    we_full = relative[:, :GP // 2]
    ve_full = relative[:, GP // 2:]

    rbig, bsn = _attention(ws_r, vs_r, we_full, ve_full)
    xmat = rbig.reshape(B, 2 * OUT, N_HASHES * H)
    spair = _cnorm(xmat)
    o_all = spair.reshape(B, N_HASHES * H, OUT, 2)[..., 0]
    ret = _combine(o_all, bsn, pos)
    ret = ret.reshape(N, W, H, OUT)
    return jnp.transpose(ret, (0, 3, 2, 1))
